# baseline (speedup 1.0000x reference)
import functools

import jax
import jax.numpy as jnp
from jax import lax
from jax.experimental import pallas as pl
from jax.experimental.pallas import tpu as pltpu
from jax.experimental.pallas import tpu_sc as plsc

_SEQ = 2048
_DIM = 768
_NL = 12
_PLD = 64
_SCALE = 8.0

_info = plsc.get_sparse_core_info()
_NC, _NS = _info.num_cores, _info.num_subcores
_NW = _NC * _NS
_B_PER_W = _SEQ // _NW
_NB = 4
_BLK = _B_PER_W // _NB

_mesh = plsc.VectorSubcoreMesh(core_axis_name="c", subcore_axis_name="s")


@functools.partial(
    pl.kernel,
    mesh=_mesh,
    out_type=jax.ShapeDtypeStruct((1, _SEQ, _NL, _PLD), jnp.float32),
    scratch_types=[
        pltpu.VMEM((_B_PER_W,), jnp.int32),
        pltpu.VMEM((_B_PER_W, _DIM), jnp.float32),
        [pltpu.VMEM((_BLK, _NL, _PLD), jnp.float32)] * 2,
        [pltpu.SemaphoreType.DMA] * _NB,
        [pltpu.SemaphoreType.DMA] * 2,
    ],
)
def _emb_gather(table_hbm, ids_hbm, out_hbm, idx_v, rows_v, out_bufs, gsems, wsems):
    wid = lax.axis_index("s") * _NC + lax.axis_index("c")
    base = wid * _B_PER_W
    pltpu.sync_copy(ids_hbm.at[pl.ds(base, _B_PER_W)], idx_v)

    gathers = []
    for b in range(_NB):
        blk = pl.ds(b * _BLK, _BLK)
        gathers.append(
            pltpu.async_copy(table_hbm.at[idx_v.at[blk]], rows_v.at[blk], gsems[b])
        )

    writes = [None, None]
    for b in range(_NB):
        gathers[b].wait()
        buf = out_bufs[b % 2]
        if writes[b % 2] is not None:
            writes[b % 2].wait()

        def scale_row(i, _):
            for l in range(_NL):
                for j in range(_PLD // 16):
                    sl = pl.ds(j * 16, 16)
                    buf[i, l, sl] = (
                        rows_v[b * _BLK + i, pl.ds(l * _PLD + j * 16, 16)] * _SCALE
                    )
            return _

        lax.fori_loop(0, _BLK, scale_row, None)
        writes[b % 2] = pltpu.async_copy(
            buf, out_hbm.at[0, pl.ds(base + b * _BLK, _BLK)], wsems[b % 2]
        )

    for w in writes:
        w.wait()


def kernel(token_ids, per_layer_table):
    ids = token_ids.reshape(-1).astype(jnp.int32)
    return _emb_gather(per_layer_table, ids)


# speedup vs baseline: 1.6172x; 1.6172x over previous
import functools

import jax
import jax.numpy as jnp
from jax import lax
from jax.experimental import pallas as pl
from jax.experimental.pallas import tpu as pltpu
from jax.experimental.pallas import tpu_sc as plsc

_SEQ = 2048
_DIM = 768
_SCALE = 8.0

_info = plsc.get_sparse_core_info()
_NC, _NS = _info.num_cores, _info.num_subcores
_NW = _NC * _NS
_B_PER_W = _SEQ // _NW
_NB = 8
_BLK = _B_PER_W // _NB

_mesh = plsc.VectorSubcoreMesh(core_axis_name="c", subcore_axis_name="s")


@functools.partial(
    pl.kernel,
    mesh=_mesh,
    out_type=jax.ShapeDtypeStruct((_SEQ, _DIM), jnp.float32),
    scratch_types=[
        pltpu.VMEM((_B_PER_W,), jnp.int32),
        pltpu.VMEM((_B_PER_W, _DIM), jnp.float32),
        [pltpu.SemaphoreType.DMA] * _NB,
        pltpu.SemaphoreType.DMA,
    ],
)
def _emb_gather(table_hbm, ids_hbm, out_hbm, idx_v, rows_v, gsems, osem):
    wid = lax.axis_index("s") * _NC + lax.axis_index("c")
    base = wid * _B_PER_W
    pltpu.sync_copy(ids_hbm.at[pl.ds(base, _B_PER_W)], idx_v)

    gathers = []
    for b in range(_NB):
        blk = pl.ds(b * _BLK, _BLK)
        gathers.append(
            pltpu.async_copy(table_hbm.at[idx_v.at[blk]], rows_v.at[blk], gsems[b])
        )

    writes = []
    for b in range(_NB):
        gathers[b].wait()

        def scale_row(i, _):
            for j in range(_DIM // 16):
                sl = pl.ds(j * 16, 16)
                rows_v[i, sl] = rows_v[i, sl] * _SCALE
            return _

        lax.fori_loop(b * _BLK, (b + 1) * _BLK, scale_row, None)
        blk = pl.ds(b * _BLK, _BLK)
        writes.append(
            pltpu.async_copy(
                rows_v.at[blk], out_hbm.at[pl.ds(base + b * _BLK, _BLK)], osem
            )
        )

    for w in writes:
        w.wait()


def kernel(token_ids, per_layer_table):
    b, s = token_ids.shape
    ids = token_ids.reshape(-1).astype(jnp.int32)
    out = _emb_gather(per_layer_table, ids)
    return out.reshape(b, s, 12, 64)


# speedup vs baseline: 1.6933x; 1.0471x over previous
import functools

import jax
import jax.numpy as jnp
from jax import lax
from jax.experimental import pallas as pl
from jax.experimental.pallas import tpu as pltpu
from jax.experimental.pallas import tpu_sc as plsc

_SEQ = 2048
_DIM = 768
_SCALE = 8.0

_info = plsc.get_sparse_core_info()
_NC, _NS = _info.num_cores, _info.num_subcores
_NW = _NC * _NS
_B_PER_W = _SEQ // _NW
_NB = 4
_BLK = _B_PER_W // _NB

_mesh = plsc.VectorSubcoreMesh(core_axis_name="c", subcore_axis_name="s")


@functools.partial(
    pl.kernel,
    mesh=_mesh,
    out_type=jax.ShapeDtypeStruct((_SEQ, _DIM), jnp.float32),
    scratch_types=[
        pltpu.VMEM((_B_PER_W,), jnp.int32),
        pltpu.VMEM((_B_PER_W, _DIM), jnp.float32),
        pltpu.SemaphoreType.DMA((_NB,)),
        pltpu.SemaphoreType.DMA,
    ],
)
def _emb_gather(table_hbm, ids_hbm, out_hbm, idx_v, rows_v, gsems, osem):
    wid = lax.axis_index("s") * _NC + lax.axis_index("c")
    base = wid * _B_PER_W
    pltpu.sync_copy(ids_hbm.at[pl.ds(base, _B_PER_W)], idx_v)

    def fire(b, _):
        blk = pl.ds(b * _BLK, _BLK)
        pltpu.async_copy(table_hbm.at[idx_v.at[blk]], rows_v.at[blk], gsems.at[b])
        return _

    lax.fori_loop(0, _NB, fire, None)

    def process(b, _):
        pltpu.make_async_copy(
            table_hbm.at[idx_v.at[pl.ds(b * _BLK, _BLK)]],
            rows_v.at[pl.ds(b * _BLK, _BLK)],
            gsems.at[b],
        ).wait()

        def scale_row(i, _):
            for j in range(_DIM // 16):
                sl = pl.ds(j * 16, 16)
                rows_v[i, sl] = rows_v[i, sl] * _SCALE
            return _

        lax.fori_loop(b * _BLK, (b + 1) * _BLK, scale_row, None)
        blk = pl.ds(b * _BLK, _BLK)
        pltpu.async_copy(
            rows_v.at[blk], out_hbm.at[pl.ds(base + b * _BLK, _BLK)], osem
        )
        return _

    lax.fori_loop(0, _NB, process, None)

    pltpu.make_async_copy(
        rows_v, out_hbm.at[pl.ds(base, _B_PER_W)], osem
    ).wait()


def kernel(token_ids, per_layer_table):
    b, s = token_ids.shape
    ids = token_ids.reshape(-1).astype(jnp.int32)
    out = _emb_gather(per_layer_table, ids)
    return out.reshape(b, s, 12, 64)
